# baseline (speedup 1.0000x reference)
import functools

import jax
import jax.numpy as jnp
from jax import lax
from jax.experimental import pallas as pl
from jax.experimental.pallas import tpu as pltpu
from jax.experimental.pallas import tpu_sc as plsc

BEV_H, BEV_W = 200, 200
NBINS = BEV_H * BEV_W
ACC = NBINS + 16
B, NUM_CAMS, C, FEAT_H, FEAT_W = 4, 6, 96, 128, 128
NPIX = FEAT_H * FEAT_W
NPAIR = B * C
NW = 32
PAIRS_PER = NPAIR // NW


def _ids_body(m_ref, y_ref, x_ref, o_ref):
    o_ref[...] = jnp.where(m_ref[...] != 0,
                           y_ref[...] * BEV_W + x_ref[...],
                           NBINS)


_ids_call = pl.pallas_call(
    _ids_body,
    out_shape=jax.ShapeDtypeStruct((NUM_CAMS * FEAT_H, FEAT_W), jnp.int32),
)


_sc_mesh = plsc.VectorSubcoreMesh(core_axis_name="c", subcore_axis_name="s")


@functools.partial(
    pl.kernel,
    mesh=_sc_mesh,
    out_type=jax.ShapeDtypeStruct((NPAIR, NBINS), jnp.float32),
    scratch_types=[
        pltpu.VMEM((ACC,), jnp.float32),
        pltpu.VMEM((NPIX,), jnp.int32),
        pltpu.VMEM((NPIX,), jnp.float32),
    ],
    compiler_params=pltpu.CompilerParams(needs_layout_passes=False,
                                         use_tc_tiling_on_sc=False),
)
def _scatter_kernel(feat_hbm, ids_hbm, out_hbm, acc, idsv, datav):
    wid = lax.axis_index("s") * 2 + lax.axis_index("c")
    zero16 = jnp.zeros((16,), jnp.float32)

    def pair_body(j, _):
        pair = wid * PAIRS_PER + j
        b = pair // C
        c = pair - b * C

        def zero_body(i, _):
            acc[pl.ds(i * 16, 16)] = zero16
            return 0

        lax.fori_loop(0, ACC // 16, zero_body, 0)

        def cam_body(cam, _):
            row = (b * NUM_CAMS + cam) * C + c
            pltpu.sync_copy(ids_hbm.at[cam], idsv)
            pltpu.sync_copy(feat_hbm.at[row], datav)

            def g_body(g, _):
                iv = idsv[pl.ds(g * 16, 16)]
                xv = datav[pl.ds(g * 16, 16)]
                plsc.addupdate_scatter(acc, [iv], xv)
                return 0

            lax.fori_loop(0, NPIX // 16, g_body, 0)
            return 0

        lax.fori_loop(0, NUM_CAMS, cam_body, 0)
        pltpu.sync_copy(acc.at[pl.ds(0, NBINS)], out_hbm.at[pair])
        return 0

    lax.fori_loop(0, PAIRS_PER, pair_body, 0)


def kernel(features, valid_masks, bev_y_indices, bev_x_indices):
    y = bev_y_indices.astype(jnp.int32).reshape(NUM_CAMS * FEAT_H, FEAT_W)
    x = bev_x_indices.astype(jnp.int32).reshape(NUM_CAMS * FEAT_H, FEAT_W)
    m = valid_masks.astype(jnp.int32).reshape(NUM_CAMS * FEAT_H, FEAT_W)
    ids = _ids_call(m, y, x).reshape(NUM_CAMS, NPIX)
    feat = features.reshape(B * NUM_CAMS * C, NPIX)
    out = _scatter_kernel(feat, ids)
    return out.reshape(B, C, BEV_H, BEV_W)


# speedup vs baseline: 1.2396x; 1.2396x over previous
import functools

import jax
import jax.numpy as jnp
from jax import lax
from jax.experimental import pallas as pl
from jax.experimental.pallas import tpu as pltpu
from jax.experimental.pallas import tpu_sc as plsc

BEV_H, BEV_W = 200, 200
NBINS = BEV_H * BEV_W
ACC = NBINS + 64
B, NUM_CAMS, C, FEAT_H, FEAT_W = 4, 6, 96, 128, 128
NPIX = FEAT_H * FEAT_W
NPAIR = B * C
NW = 32
PAIRS_PER = NPAIR // NW


def _ids_body(m_ref, y_ref, x_ref, o_ref):
    o_ref[...] = jnp.where(m_ref[...] != 0,
                           y_ref[...] * BEV_W + x_ref[...],
                           NBINS)


_ids_call = pl.pallas_call(
    _ids_body,
    out_shape=jax.ShapeDtypeStruct((NUM_CAMS * FEAT_H, FEAT_W), jnp.int32),
)


_sc_mesh = plsc.VectorSubcoreMesh(core_axis_name="c", subcore_axis_name="s")


@functools.partial(
    pl.kernel,
    mesh=_sc_mesh,
    out_type=jax.ShapeDtypeStruct((NPAIR, NBINS), jnp.float32),
    scratch_types=[
        pltpu.VMEM((ACC,), jnp.float32),
        pltpu.VMEM((NPIX,), jnp.int32),
        pltpu.VMEM((NPIX,), jnp.float32),
    ],
    compiler_params=pltpu.CompilerParams(needs_layout_passes=False,
                                         use_tc_tiling_on_sc=False),
)
def _scatter_kernel(feat_hbm, ids_hbm, out_hbm, acc, idsv, datav):
    wid = lax.axis_index("s") * 2 + lax.axis_index("c")
    zero16 = jnp.zeros((16,), jnp.float32)

    def pair_body(j, _):
        pair = wid * PAIRS_PER + j
        b = pair // C
        c = pair - b * C

        @plsc.parallel_loop(0, ACC // 16, 1, unroll=8)
        def zero_body(i):
            acc[pl.ds(i * 16, 16)] = zero16

        def cam_body(cam, _):
            row = (b * NUM_CAMS + cam) * C + c
            pltpu.sync_copy(ids_hbm.at[cam], idsv)
            pltpu.sync_copy(feat_hbm.at[row], datav)

            @plsc.parallel_loop(0, NPIX // 16, 1, unroll=8)
            def g_body(g):
                iv = idsv[pl.ds(g * 16, 16)]
                xv = datav[pl.ds(g * 16, 16)]
                plsc.addupdate_scatter(acc, [iv], xv)

            return 0

        lax.fori_loop(0, NUM_CAMS, cam_body, 0)
        pltpu.sync_copy(acc.at[pl.ds(0, NBINS)], out_hbm.at[pair])
        return 0

    lax.fori_loop(0, PAIRS_PER, pair_body, 0)


def kernel(features, valid_masks, bev_y_indices, bev_x_indices):
    y = bev_y_indices.astype(jnp.int32).reshape(NUM_CAMS * FEAT_H, FEAT_W)
    x = bev_x_indices.astype(jnp.int32).reshape(NUM_CAMS * FEAT_H, FEAT_W)
    m = valid_masks.astype(jnp.int32).reshape(NUM_CAMS * FEAT_H, FEAT_W)
    ids = _ids_call(m, y, x).reshape(NUM_CAMS, NPIX)
    feat = features.reshape(B * NUM_CAMS * C, NPIX)
    out = _scatter_kernel(feat, ids)
    return out.reshape(B, C, BEV_H, BEV_W)


# speedup vs baseline: 1.4777x; 1.1920x over previous
import functools

import jax
import jax.numpy as jnp
from jax import lax
from jax.experimental import pallas as pl
from jax.experimental.pallas import tpu as pltpu
from jax.experimental.pallas import tpu_sc as plsc

BEV_H, BEV_W = 200, 200
NBINS = BEV_H * BEV_W
ACC = NBINS + 64
B, NUM_CAMS, C, FEAT_H, FEAT_W = 4, 6, 96, 128, 128
NPIX = FEAT_H * FEAT_W
NPAIR = B * C
NW = 32
PAIRS_PER = NPAIR // NW


def _ids_body(m_ref, y_ref, x_ref, o_ref):
    o_ref[...] = jnp.where(m_ref[...] != 0,
                           y_ref[...] * BEV_W + x_ref[...],
                           NBINS)


_ids_call = pl.pallas_call(
    _ids_body,
    out_shape=jax.ShapeDtypeStruct((NUM_CAMS * FEAT_H, FEAT_W), jnp.int32),
)


_sc_mesh = plsc.VectorSubcoreMesh(core_axis_name="c", subcore_axis_name="s")


@functools.partial(
    pl.kernel,
    mesh=_sc_mesh,
    out_type=jax.ShapeDtypeStruct((NPAIR, NBINS), jnp.float32),
    scratch_types=[
        pltpu.VMEM((ACC,), jnp.float32),
        pltpu.VMEM((2, NPIX), jnp.int32),
        pltpu.VMEM((2, NPIX), jnp.float32),
        pltpu.SemaphoreType.DMA,
        pltpu.SemaphoreType.DMA,
        pltpu.SemaphoreType.DMA,
        pltpu.SemaphoreType.DMA,
    ],
    compiler_params=pltpu.CompilerParams(needs_layout_passes=False,
                                         use_tc_tiling_on_sc=False),
)
def _scatter_kernel(feat_hbm, ids_hbm, out_hbm, acc, idsv, datav,
                    sem_i0, sem_i1, sem_d0, sem_d1):
    wid = lax.axis_index("s") * 2 + lax.axis_index("c")
    zero16 = jnp.zeros((16,), jnp.float32)
    sem_i = (sem_i0, sem_i1)
    sem_d = (sem_d0, sem_d1)

    def pair_body(j, _):
        pair = wid * PAIRS_PER + j
        b = pair // C
        c = pair - b * C

        def issue(cam, slot):
            row = (b * NUM_CAMS + cam) * C + c
            hi = pltpu.async_copy(ids_hbm.at[cam], idsv.at[slot], sem_i[slot])
            hd = pltpu.async_copy(feat_hbm.at[row], datav.at[slot], sem_d[slot])
            return hi, hd

        hs = [None, None]
        hs[0] = issue(0, 0)

        @plsc.parallel_loop(0, ACC // 16, 1, unroll=8)
        def zero_body(i):
            acc[pl.ds(i * 16, 16)] = zero16

        for cam in range(NUM_CAMS):
            slot = cam % 2
            if cam + 1 < NUM_CAMS:
                hs[(cam + 1) % 2] = issue(cam + 1, (cam + 1) % 2)
            hi, hd = hs[slot]
            hi.wait()
            hd.wait()

            @plsc.parallel_loop(0, NPIX // 16, 1, unroll=8)
            def g_body(g):
                iv = idsv[slot, pl.ds(g * 16, 16)]
                xv = datav[slot, pl.ds(g * 16, 16)]
                plsc.addupdate_scatter(acc, [iv], xv)

        pltpu.sync_copy(acc.at[pl.ds(0, NBINS)], out_hbm.at[pair])
        return 0

    lax.fori_loop(0, PAIRS_PER, pair_body, 0)


def kernel(features, valid_masks, bev_y_indices, bev_x_indices):
    y = bev_y_indices.astype(jnp.int32).reshape(NUM_CAMS * FEAT_H, FEAT_W)
    x = bev_x_indices.astype(jnp.int32).reshape(NUM_CAMS * FEAT_H, FEAT_W)
    m = valid_masks.astype(jnp.int32).reshape(NUM_CAMS * FEAT_H, FEAT_W)
    ids = _ids_call(m, y, x).reshape(NUM_CAMS, NPIX)
    feat = features.reshape(B * NUM_CAMS * C, NPIX)
    out = _scatter_kernel(feat, ids)
    return out.reshape(B, C, BEV_H, BEV_W)
